# baseline (speedup 1.0000x reference)
import functools
import math

import jax
import jax.numpy as jnp
from jax import lax
from jax.experimental import pallas as pl
from jax.experimental.pallas import tpu as pltpu
from jax.experimental.pallas import tpu_sc as plsc

VOCAB = 1000000
DIM = 64
BATCH = 16384
HIST = 50
SCALE = math.sqrt(DIM)

_info = plsc.get_sparse_core_info()
NC = _info.num_cores
NS = _info.num_subcores
NW = NC * NS
B_TOTAL = BATCH * HIST
B_PER_W = B_TOTAL // NW
CHUNK = 512
N_CHUNKS = B_PER_W // CHUNK

_mesh = plsc.VectorSubcoreMesh(core_axis_name="c", subcore_axis_name="s")


@functools.partial(
    pl.kernel,
    mesh=_mesh,
    out_type=jax.ShapeDtypeStruct((B_TOTAL, DIM), jnp.float32),
    scratch_types=[
        pltpu.VMEM((CHUNK,), jnp.int32),
        pltpu.VMEM((CHUNK, DIM), jnp.float32),
        pltpu.SemaphoreType.DMA,
    ],
    compiler_params=pltpu.CompilerParams(use_tc_tiling_on_sc=False),
)
def _embed_sc(idx_hbm, table_hbm, out_hbm, idx_v, rows_v, sem):
    wid = lax.axis_index("s") * NC + lax.axis_index("c")
    base = wid * B_PER_W

    def chunk_body(i, carry):
        off = base + i * CHUNK
        pltpu.sync_copy(idx_hbm.at[pl.ds(off, CHUNK)], idx_v)
        pltpu.async_copy(table_hbm.at[idx_v], rows_v, sem).wait()

        def scale_body(j, c2):
            for k in range(DIM // 16):
                sl = (j, pl.ds(k * 16, 16))
                rows_v[sl] = rows_v[sl] * SCALE
            return c2

        lax.fori_loop(0, CHUNK, scale_body, 0)
        pltpu.sync_copy(rows_v, out_hbm.at[pl.ds(off, CHUNK)])
        return carry

    lax.fori_loop(0, N_CHUNKS, chunk_body, 0)


def kernel(input, table):
    idx = input.reshape(-1).astype(jnp.int32)
    out = _embed_sc(idx, table)
    return out.reshape(BATCH, HIST, DIM)


# speedup vs baseline: 1.1338x; 1.1338x over previous
import functools
import math

import jax
import jax.numpy as jnp
from jax import lax
from jax.experimental import pallas as pl
from jax.experimental.pallas import tpu as pltpu
from jax.experimental.pallas import tpu_sc as plsc

VOCAB = 1000000
DIM = 64
BATCH = 16384
HIST = 50
SCALE = math.sqrt(DIM)

_info = plsc.get_sparse_core_info()
NC = _info.num_cores
NS = _info.num_subcores
NW = NC * NS
B_TOTAL = BATCH * HIST
B_PER_W = B_TOTAL // NW
CHUNK = 640
N_CHUNKS = B_PER_W // CHUNK

_mesh = plsc.VectorSubcoreMesh(core_axis_name="c", subcore_axis_name="s")


@functools.partial(
    pl.kernel,
    mesh=_mesh,
    out_type=jax.ShapeDtypeStruct((B_TOTAL, DIM), jnp.float32),
    scratch_types=[
        pltpu.VMEM((B_PER_W,), jnp.int32),
        pltpu.VMEM((CHUNK, DIM), jnp.float32),
        pltpu.VMEM((CHUNK, DIM), jnp.float32),
        pltpu.SemaphoreType.DMA,
        pltpu.SemaphoreType.DMA,
        pltpu.SemaphoreType.DMA,
        pltpu.SemaphoreType.DMA,
    ],
    compiler_params=pltpu.CompilerParams(use_tc_tiling_on_sc=False),
)
def _embed_sc(idx_hbm, table_hbm, out_hbm, idx_v, rows0, rows1, g0, g1, s0, s1):
    wid = lax.axis_index("s") * NC + lax.axis_index("c")
    base = wid * B_PER_W
    bufs = (rows0, rows1)
    gsems = (g0, g1)
    ssems = (s0, s1)

    pltpu.sync_copy(idx_hbm.at[pl.ds(base, B_PER_W)], idx_v)

    def start_gather(i, b):
        pltpu.async_copy(table_hbm.at[idx_v.at[pl.ds(i * CHUNK, CHUNK)]],
                         bufs[b], gsems[b])

    def scale_buf(b):
        buf = bufs[b]

        def row_body(j, c):
            for k in range(DIM // 16):
                sl = (j, pl.ds(k * 16, 16))
                buf[sl] = buf[sl] * SCALE
            return c

        lax.fori_loop(0, CHUNK, row_body, 0, unroll=2)

    start_gather(0, 0)

    def pair_body(t, carry):
        for b in range(2):
            i = t * 2 + b
            nb = 1 - b

            @pl.when(i + 1 < N_CHUNKS)
            def _():
                @pl.when(i >= 1)
                def _():
                    pltpu.make_async_copy(
                        bufs[nb], out_hbm.at[pl.ds(base, CHUNK)], ssems[nb]
                    ).wait()

                start_gather(i + 1, nb)

            pltpu.make_async_copy(
                table_hbm.at[idx_v.at[pl.ds(i * CHUNK, CHUNK)]], bufs[b], gsems[b]
            ).wait()
            scale_buf(b)
            pltpu.async_copy(bufs[b], out_hbm.at[pl.ds(base + i * CHUNK, CHUNK)],
                             ssems[b])
        return carry

    lax.fori_loop(0, N_CHUNKS // 2, pair_body, 0)

    pltpu.make_async_copy(bufs[0], out_hbm.at[pl.ds(base, CHUNK)], ssems[0]).wait()
    pltpu.make_async_copy(bufs[1], out_hbm.at[pl.ds(base, CHUNK)], ssems[1]).wait()


def kernel(input, table):
    idx = input.reshape(-1).astype(jnp.int32)
    out = _embed_sc(idx, table)
    return out.reshape(BATCH, HIST, DIM)
